# baseline (speedup 1.0000x reference)
import functools

import jax
import jax.numpy as jnp
from jax import lax
from jax.experimental import pallas as pl
from jax.experimental.pallas import tpu as pltpu
from jax.experimental.pallas import tpu_sc as plsc

MARGIN = 1.0
NC, NS, L = 2, 16, 16
NW = NC * NS
CH = 64


def _triplet_body(emb_hbm, ai_hbm, pi_hbm, ni_hbm, out_hbm,
                  ia0, ip0, in0, ia1, ip1, in1,
                  xa0, xp0, xn0, xa1, xp1, xn1,
                  pa0, pp0, pn0, pa1, pp1, pn1,
                  ra0, rp0, rn0, ra1, rp1, rn1,
                  tot_v, sem0, sem1, isem0, isem1):
    T = ai_hbm.shape[0]
    per_w = T // NW
    n_chunks = per_w // CH
    D2 = emb_hbm.shape[2]
    D = D2 // 2
    G = CH // L

    wid = lax.axis_index("s") * NC + lax.axis_index("c")
    base = wid * per_w

    lane = lax.iota(jnp.int32, L)
    zero = jnp.zeros((L,), jnp.int32)
    bufs = ((ia0, ip0, in0, xa0, xp0, xn0, pa0, pp0, pn0,
             ra0, rp0, rn0, sem0, isem0),
            (ia1, ip1, in1, xa1, xp1, xn1, pa1, pp1, pn1,
             ra1, rp1, rn1, sem1, isem1))

    def stage_idx(c, parity):
        ia, ip, in_ = bufs[parity][0:3]
        isem = bufs[parity][13]
        pltpu.async_copy(ai_hbm.at[pl.ds(base + c * CH, CH)], ia, isem)
        pltpu.async_copy(pi_hbm.at[pl.ds(base + c * CH, CH)], ip, isem)
        pltpu.async_copy(ni_hbm.at[pl.ds(base + c * CH, CH)], in_, isem)

    def issue(c, parity):
        (ia, ip, in_, xa, xp, xn, pa, pp, pn,
         ra, rp, rn, sem, isem) = bufs[parity]
        pltpu.make_async_copy(
            ai_hbm.at[pl.ds(base + c * CH, CH)], ia, isem).wait()
        pltpu.make_async_copy(
            pi_hbm.at[pl.ds(base + c * CH, CH)], ip, isem).wait()
        pltpu.make_async_copy(
            ni_hbm.at[pl.ds(base + c * CH, CH)], in_, isem).wait()
        for src_v, dst_v, par_v in ((ia, xa, pa), (ip, xp, pp), (in_, xn, pn)):
            for k in range(G):
                pos = k * L + lane
                v = plsc.load_gather(src_v, [pos])
                plsc.store_scatter(dst_v, [pos], v >> 1)
                plsc.store_scatter(par_v, [pos], (v & 1) * D)
        pltpu.async_copy(emb_hbm.at[xa], ra, sem)
        pltpu.async_copy(emb_hbm.at[xp], rp, sem)
        pltpu.async_copy(emb_hbm.at[xn], rn, sem)

        @pl.when(c + 2 < n_chunks)
        def _():
            stage_idx(c + 2, parity)

    def wait(parity):
        xa, xp, xn = bufs[parity][3:6]
        ra, rp, rn = bufs[parity][9:12]
        sem = bufs[parity][12]
        pltpu.make_async_copy(emb_hbm.at[xa], ra, sem).wait()
        pltpu.make_async_copy(emb_hbm.at[xp], rp, sem).wait()
        pltpu.make_async_copy(emb_hbm.at[xn], rn, sem).wait()

    def compute(parity, total):
        pa, pp, pn = bufs[parity][6:9]
        ra, rp, rn = bufs[parity][9:12]

        def group_body(g, tot):
            rows = g * L + lane
            oa = plsc.load_gather(pa, [rows])
            op = plsc.load_gather(pp, [rows])
            on = plsc.load_gather(pn, [rows])
            acc = jnp.zeros((L,), jnp.float32)
            for d in range(D):
                sk = (lane + d) & (D - 1)
                va = plsc.load_gather(ra, [rows, zero, oa + sk])
                vp = plsc.load_gather(rp, [rows, zero, op + sk])
                vn = plsc.load_gather(rn, [rows, zero, on + sk])
                t1 = va - vp
                t2 = va - vn
                acc = acc + (t1 * t1 - t2 * t2)
            return tot + jnp.maximum(acc + MARGIN, 0.0)

        return lax.fori_loop(0, G, group_body, total)

    stage_idx(0, 0)
    stage_idx(1, 1)
    issue(0, 0)

    def pair_body(h, total):
        c0 = 2 * h
        issue(c0 + 1, 1)
        wait(0)
        total = compute(0, total)

        @pl.when(c0 + 2 < n_chunks)
        def _():
            issue(c0 + 2, 0)
        wait(1)
        return compute(1, total)

    total = lax.fori_loop(0, n_chunks // 2, pair_body,
                          jnp.zeros((L,), jnp.float32))
    tot_v[...] = total
    pltpu.sync_copy(tot_v, out_hbm.at[wid])


def kernel(embeddings, target, triplets):
    del target
    T = triplets.shape[0]
    B, D = embeddings.shape
    per_w = T // NW
    ai = triplets[:, 0]
    pi = triplets[:, 1]
    ni = triplets[:, 2]

    emb_pair = embeddings.reshape(B // 2, 1, 2 * D)

    f = pl.kernel(
        _triplet_body,
        out_type=jax.ShapeDtypeStruct((NW, L), jnp.float32),
        mesh=plsc.VectorSubcoreMesh(core_axis_name="c", subcore_axis_name="s"),
        compiler_params=pltpu.CompilerParams(needs_layout_passes=False),
        scratch_types=(
            [pltpu.VMEM((CH,), jnp.int32)] * 18
            + [pltpu.VMEM((CH, 1, 2 * D), jnp.float32)] * 6
            + [pltpu.VMEM((L,), jnp.float32),
               pltpu.SemaphoreType.DMA,
               pltpu.SemaphoreType.DMA,
               pltpu.SemaphoreType.DMA,
               pltpu.SemaphoreType.DMA]
        ),
    )
    partials = f(emb_pair, ai, pi, ni)
    loss = jnp.sum(partials) / T
    return (loss, T, T)
